# Initial kernel scaffold; baseline (speedup 1.0000x reference)
import jax
import jax.numpy as jnp
from jax.experimental import pallas as pl


def kernel(inputs, seq_len, W_ih_f, W_hh_f, b_ih_f, b_hh_f, W_ih_b, W_hh_b, b_ih_b, b_hh_b):
    raise NotImplementedError("write your pallas kernel here")


# speedup vs baseline: 174.3582x; 174.3582x over previous
import jax
import jax.numpy as jnp
from jax.experimental import pallas as pl
from jax.experimental.pallas import tpu as pltpu

_TC = 128

_PREC = jax.lax.Precision.HIGHEST


def _bilstm_kernel(seq_ref, xf_ref, xb_ref,
                   wih_f_ref, whh_f_ref, b_f_ref,
                   wih_b_ref, whh_b_ref, b_b_ref,
                   outf_ref, outb_ref,
                   gf_ref, gb_ref, hf_ref, cf_ref, hb_ref, cb_ref):
    k = pl.program_id(0)
    nchunk = pl.num_programs(0)
    tc, bsz, d = xf_ref.shape
    h_dim = whh_f_ref.shape[0]

    @pl.when(k == 0)
    def _init():
        hf_ref[...] = jnp.zeros_like(hf_ref)
        cf_ref[...] = jnp.zeros_like(cf_ref)
        hb_ref[...] = jnp.zeros_like(hb_ref)
        cb_ref[...] = jnp.zeros_like(cb_ref)

    xf = xf_ref[...].reshape(tc * bsz, d)
    gf_ref[...] = (jnp.dot(xf, wih_f_ref[...], preferred_element_type=jnp.float32,
                           precision=_PREC) + b_f_ref[...]).reshape(tc, bsz, 4 * h_dim)
    xb = xb_ref[...].reshape(tc * bsz, d)
    gb_ref[...] = (jnp.dot(xb, wih_b_ref[...], preferred_element_type=jnp.float32,
                           precision=_PREC) + b_b_ref[...]).reshape(tc, bsz, 4 * h_dim)

    seq_b = seq_ref[...]
    whh_f = whh_f_ref[...]
    whh_b = whh_b_ref[...]

    t0f = k * tc
    t0b = (nchunk - 1 - k) * tc

    def cell(gate_row, h, c, whh):
        gates = gate_row + jnp.dot(h, whh, preferred_element_type=jnp.float32,
                                   precision=_PREC)
        ig = jax.nn.sigmoid(gates[:, 0:h_dim])
        fg = jax.nn.sigmoid(gates[:, h_dim:2 * h_dim])
        gg = jnp.tanh(gates[:, 2 * h_dim:3 * h_dim])
        og = jax.nn.sigmoid(gates[:, 3 * h_dim:4 * h_dim])
        c = fg * c + ig * gg
        h = og * jnp.tanh(c)
        return h, c

    def body(i, carry):
        hf, cf, hb, cb = carry
        tf = t0f + i
        hf, cf = cell(gf_ref[i], hf, cf, whh_f)
        outf_ref[i] = jnp.where(tf < seq_b, hf, 0.0)
        j = tc - 1 - i
        tb = t0b + j
        keep = tb < seq_b - 1
        hb_in = jnp.where(keep, hb, 0.0)
        cb_in = jnp.where(keep, cb, 0.0)
        hb, cb = cell(gb_ref[j], hb_in, cb_in, whh_b)
        outb_ref[j] = jnp.where(tb < seq_b, hb, 0.0)
        return hf, cf, hb, cb

    carry0 = (hf_ref[...], cf_ref[...], hb_ref[...], cb_ref[...])
    hf, cf, hb, cb = jax.lax.fori_loop(0, tc, body, carry0)
    hf_ref[...] = hf
    cf_ref[...] = cf
    hb_ref[...] = hb
    cb_ref[...] = cb


def kernel(inputs, seq_len, W_ih_f, W_hh_f, b_ih_f, b_hh_f,
           W_ih_b, W_hh_b, b_ih_b, b_hh_b):
    bsz, t_len, d = inputs.shape
    h_dim = W_hh_f.shape[1]
    tc = min(_TC, t_len)
    nchunk = t_len // tc
    assert t_len % tc == 0

    xT = jnp.transpose(inputs, (1, 0, 2))
    seq_b = jnp.broadcast_to(seq_len.astype(jnp.int32)[:, None], (bsz, h_dim))
    args = (seq_b, xT, xT,
            W_ih_f.T, W_hh_f.T, (b_ih_f + b_hh_f)[None, :],
            W_ih_b.T, W_hh_b.T, (b_ih_b + b_hh_b)[None, :])

    full2 = lambda k: (0, 0)
    in_specs = [
        pl.BlockSpec((bsz, h_dim), full2),
        pl.BlockSpec((tc, bsz, d), lambda k: (k, 0, 0)),
        pl.BlockSpec((tc, bsz, d), lambda k: (nchunk - 1 - k, 0, 0)),
        pl.BlockSpec((d, 4 * h_dim), full2),
        pl.BlockSpec((h_dim, 4 * h_dim), full2),
        pl.BlockSpec((1, 4 * h_dim), full2),
        pl.BlockSpec((d, 4 * h_dim), full2),
        pl.BlockSpec((h_dim, 4 * h_dim), full2),
        pl.BlockSpec((1, 4 * h_dim), full2),
    ]
    out_specs = [
        pl.BlockSpec((tc, bsz, h_dim), lambda k: (k, 0, 0)),
        pl.BlockSpec((tc, bsz, h_dim), lambda k: (nchunk - 1 - k, 0, 0)),
    ]
    outf, outb = pl.pallas_call(
        _bilstm_kernel,
        grid=(nchunk,),
        in_specs=in_specs,
        out_specs=out_specs,
        out_shape=[jax.ShapeDtypeStruct((t_len, bsz, h_dim), jnp.float32)] * 2,
        scratch_shapes=[
            pltpu.VMEM((tc, bsz, 4 * h_dim), jnp.float32),
            pltpu.VMEM((tc, bsz, 4 * h_dim), jnp.float32),
            pltpu.VMEM((bsz, h_dim), jnp.float32),
            pltpu.VMEM((bsz, h_dim), jnp.float32),
            pltpu.VMEM((bsz, h_dim), jnp.float32),
            pltpu.VMEM((bsz, h_dim), jnp.float32),
        ],
    )(*args)
    out = jnp.concatenate([outf, outb], axis=-1)
    return jnp.transpose(out, (1, 0, 2))


# speedup vs baseline: 695.5032x; 3.9889x over previous
import jax
import jax.numpy as jnp
from jax.experimental import pallas as pl
from jax.experimental.pallas import tpu as pltpu

_TC = 128


def _bilstm_kernel(seq_ref, xf_ref, xb_ref,
                   wih_f_ref, whh_f_ref, b_f_ref,
                   wih_b_ref, whh_b_ref, b_b_ref,
                   outf_ref, outb_ref,
                   gf_ref, gb_ref, hf_ref, cf_ref, hb_ref, cb_ref):
    k = pl.program_id(0)
    nchunk = pl.num_programs(0)
    tc, bsz, d = xf_ref.shape
    h_dim = whh_f_ref.shape[0]

    @pl.when(k == 0)
    def _init():
        hf_ref[...] = jnp.zeros_like(hf_ref)
        cf_ref[...] = jnp.zeros_like(cf_ref)
        hb_ref[...] = jnp.zeros_like(hb_ref)
        cb_ref[...] = jnp.zeros_like(cb_ref)

    xf = xf_ref[...].reshape(tc * bsz, d)
    gf_ref[...] = (jnp.dot(xf, wih_f_ref[...], preferred_element_type=jnp.float32)
                   + b_f_ref[...]).reshape(tc, bsz, 4 * h_dim)
    xb = xb_ref[...].reshape(tc * bsz, d)
    gb_ref[...] = (jnp.dot(xb, wih_b_ref[...], preferred_element_type=jnp.float32)
                   + b_b_ref[...]).reshape(tc, bsz, 4 * h_dim)

    seq_b = seq_ref[...]
    whh_f = whh_f_ref[...]
    whh_b = whh_b_ref[...]

    t0f = k * tc
    t0b = (nchunk - 1 - k) * tc

    def cell(gate_row, h, c, whh):
        gates = gate_row + jnp.dot(h, whh, preferred_element_type=jnp.float32)
        sig = jax.nn.sigmoid(gates[:, 0:3 * h_dim])
        ig = sig[:, 0:h_dim]
        fg = sig[:, h_dim:2 * h_dim]
        og = sig[:, 2 * h_dim:3 * h_dim]
        gg = jnp.tanh(gates[:, 3 * h_dim:4 * h_dim])
        c = fg * c + ig * gg
        h = og * jnp.tanh(c)
        return h, c

    def body(i, carry):
        hf, cf, hb, cb = carry
        tf = t0f + i
        hf, cf = cell(gf_ref[i], hf, cf, whh_f)
        outf_ref[i] = jnp.where(tf < seq_b, hf, 0.0)
        j = tc - 1 - i
        tb = t0b + j
        keep = tb < seq_b - 1
        hb_in = jnp.where(keep, hb, 0.0)
        cb_in = jnp.where(keep, cb, 0.0)
        hb, cb = cell(gb_ref[j], hb_in, cb_in, whh_b)
        outb_ref[j] = jnp.where(tb < seq_b, hb, 0.0)
        return hf, cf, hb, cb

    carry0 = (hf_ref[...], cf_ref[...], hb_ref[...], cb_ref[...])
    hf, cf, hb, cb = jax.lax.fori_loop(0, tc, body, carry0, unroll=2)
    hf_ref[...] = hf
    cf_ref[...] = cf
    hb_ref[...] = hb
    cb_ref[...] = cb


def kernel(inputs, seq_len, W_ih_f, W_hh_f, b_ih_f, b_hh_f,
           W_ih_b, W_hh_b, b_ih_b, b_hh_b):
    bsz, t_len, d = inputs.shape
    h_dim = W_hh_f.shape[1]
    tc = min(_TC, t_len)
    nchunk = t_len // tc
    assert t_len % tc == 0

    xT = jnp.transpose(inputs, (1, 0, 2))
    seq_b = jnp.broadcast_to(seq_len.astype(jnp.int32)[:, None], (bsz, h_dim))

    def perm(w):
        return jnp.concatenate([w[0:h_dim], w[h_dim:2 * h_dim],
                                w[3 * h_dim:4 * h_dim], w[2 * h_dim:3 * h_dim]],
                               axis=0)

    args = (seq_b, xT, xT,
            perm(W_ih_f).T, perm(W_hh_f).T, perm(b_ih_f + b_hh_f)[None, :],
            perm(W_ih_b).T, perm(W_hh_b).T, perm(b_ih_b + b_hh_b)[None, :])

    full2 = lambda k: (0, 0)
    in_specs = [
        pl.BlockSpec((bsz, h_dim), full2),
        pl.BlockSpec((tc, bsz, d), lambda k: (k, 0, 0)),
        pl.BlockSpec((tc, bsz, d), lambda k: (nchunk - 1 - k, 0, 0)),
        pl.BlockSpec((d, 4 * h_dim), full2),
        pl.BlockSpec((h_dim, 4 * h_dim), full2),
        pl.BlockSpec((1, 4 * h_dim), full2),
        pl.BlockSpec((d, 4 * h_dim), full2),
        pl.BlockSpec((h_dim, 4 * h_dim), full2),
        pl.BlockSpec((1, 4 * h_dim), full2),
    ]
    out_specs = [
        pl.BlockSpec((tc, bsz, h_dim), lambda k: (k, 0, 0)),
        pl.BlockSpec((tc, bsz, h_dim), lambda k: (nchunk - 1 - k, 0, 0)),
    ]
    outf, outb = pl.pallas_call(
        _bilstm_kernel,
        grid=(nchunk,),
        in_specs=in_specs,
        out_specs=out_specs,
        out_shape=[jax.ShapeDtypeStruct((t_len, bsz, h_dim), jnp.float32)] * 2,
        scratch_shapes=[
            pltpu.VMEM((tc, bsz, 4 * h_dim), jnp.float32),
            pltpu.VMEM((tc, bsz, 4 * h_dim), jnp.float32),
            pltpu.VMEM((bsz, h_dim), jnp.float32),
            pltpu.VMEM((bsz, h_dim), jnp.float32),
            pltpu.VMEM((bsz, h_dim), jnp.float32),
            pltpu.VMEM((bsz, h_dim), jnp.float32),
        ],
    )(*args)
    out = jnp.concatenate([outf, outb], axis=-1)
    return jnp.transpose(out, (1, 0, 2))


# speedup vs baseline: 763.1788x; 1.0973x over previous
import jax
import jax.numpy as jnp
from jax.experimental import pallas as pl
from jax.experimental.pallas import tpu as pltpu

_TC = 128


def _bilstm_kernel(seq_ref, xf_ref, xb_ref,
                   wih_f_ref, whh_f_ref, b_f_ref,
                   wih_b_ref, whh_b_ref, b_b_ref,
                   outf_ref, outb_ref,
                   gf_ref, gb_ref, hf_ref, cf_ref, hb_ref, cb_ref):
    k = pl.program_id(0)
    nchunk = pl.num_programs(0)
    tc, bsz, d = xf_ref.shape
    h_dim = whh_f_ref.shape[0]

    @pl.when(k == 0)
    def _init():
        hf_ref[...] = jnp.zeros_like(hf_ref)
        cf_ref[...] = jnp.zeros_like(cf_ref)
        hb_ref[...] = jnp.zeros_like(hb_ref)
        cb_ref[...] = jnp.zeros_like(cb_ref)

    xf = xf_ref[...].reshape(tc * bsz, d)
    gf_ref[...] = (jnp.dot(xf, wih_f_ref[...], preferred_element_type=jnp.float32)
                   + b_f_ref[...]).reshape(tc, bsz, 4 * h_dim)
    xb = xb_ref[...].reshape(tc * bsz, d)
    gb_ref[...] = (jnp.dot(xb, wih_b_ref[...], preferred_element_type=jnp.float32)
                   + b_b_ref[...]).reshape(tc, bsz, 4 * h_dim)

    seq_b = seq_ref[...]
    whh_f = whh_f_ref[...]
    whh_b = whh_b_ref[...]

    t0f = k * tc
    t0b = (nchunk - 1 - k) * tc

    def cell(gate_row, h, c, whh):
        gates = gate_row + jnp.dot(h, whh, preferred_element_type=jnp.float32)
        sig = jax.nn.sigmoid(gates[:, 0:3 * h_dim])
        ig = sig[:, 0:h_dim]
        fg = sig[:, h_dim:2 * h_dim]
        og = sig[:, 2 * h_dim:3 * h_dim]
        gg = jnp.tanh(gates[:, 3 * h_dim:4 * h_dim])
        c = fg * c + ig * gg
        h = og * jnp.tanh(c)
        return h, c

    def body(i, carry):
        hf, cf, hb, cb = carry
        tf = t0f + i
        hf, cf = cell(gf_ref[i], hf, cf, whh_f)
        outf_ref[i] = jnp.where(tf < seq_b, hf, 0.0)
        j = tc - 1 - i
        tb = t0b + j
        keep = tb < seq_b - 1
        hb_in = jnp.where(keep, hb, 0.0)
        cb_in = jnp.where(keep, cb, 0.0)
        hb, cb = cell(gb_ref[j], hb_in, cb_in, whh_b)
        outb_ref[j] = jnp.where(tb < seq_b, hb, 0.0)
        return hf, cf, hb, cb

    carry0 = (hf_ref[...], cf_ref[...], hb_ref[...], cb_ref[...])
    hf, cf, hb, cb = jax.lax.fori_loop(0, tc, body, carry0, unroll=4)
    hf_ref[...] = hf
    cf_ref[...] = cf
    hb_ref[...] = hb
    cb_ref[...] = cb


def kernel(inputs, seq_len, W_ih_f, W_hh_f, b_ih_f, b_hh_f,
           W_ih_b, W_hh_b, b_ih_b, b_hh_b):
    bsz, t_len, d = inputs.shape
    h_dim = W_hh_f.shape[1]
    tc = min(_TC, t_len)
    nchunk = t_len // tc
    assert t_len % tc == 0

    xT = jnp.transpose(inputs, (1, 0, 2))
    seq_b = jnp.broadcast_to(seq_len.astype(jnp.int32)[:, None], (bsz, h_dim))

    def perm(w):
        return jnp.concatenate([w[0:h_dim], w[h_dim:2 * h_dim],
                                w[3 * h_dim:4 * h_dim], w[2 * h_dim:3 * h_dim]],
                               axis=0)

    args = (seq_b, xT, xT,
            perm(W_ih_f).T, perm(W_hh_f).T, perm(b_ih_f + b_hh_f)[None, :],
            perm(W_ih_b).T, perm(W_hh_b).T, perm(b_ih_b + b_hh_b)[None, :])

    full2 = lambda k: (0, 0)
    in_specs = [
        pl.BlockSpec((bsz, h_dim), full2),
        pl.BlockSpec((tc, bsz, d), lambda k: (k, 0, 0)),
        pl.BlockSpec((tc, bsz, d), lambda k: (nchunk - 1 - k, 0, 0)),
        pl.BlockSpec((d, 4 * h_dim), full2),
        pl.BlockSpec((h_dim, 4 * h_dim), full2),
        pl.BlockSpec((1, 4 * h_dim), full2),
        pl.BlockSpec((d, 4 * h_dim), full2),
        pl.BlockSpec((h_dim, 4 * h_dim), full2),
        pl.BlockSpec((1, 4 * h_dim), full2),
    ]
    out_specs = [
        pl.BlockSpec((tc, bsz, h_dim), lambda k: (k, 0, 0)),
        pl.BlockSpec((tc, bsz, h_dim), lambda k: (nchunk - 1 - k, 0, 0)),
    ]
    outf, outb = pl.pallas_call(
        _bilstm_kernel,
        grid=(nchunk,),
        in_specs=in_specs,
        out_specs=out_specs,
        out_shape=[jax.ShapeDtypeStruct((t_len, bsz, h_dim), jnp.float32)] * 2,
        scratch_shapes=[
            pltpu.VMEM((tc, bsz, 4 * h_dim), jnp.float32),
            pltpu.VMEM((tc, bsz, 4 * h_dim), jnp.float32),
            pltpu.VMEM((bsz, h_dim), jnp.float32),
            pltpu.VMEM((bsz, h_dim), jnp.float32),
            pltpu.VMEM((bsz, h_dim), jnp.float32),
            pltpu.VMEM((bsz, h_dim), jnp.float32),
        ],
    )(*args)
    out = jnp.concatenate([outf, outb], axis=-1)
    return jnp.transpose(out, (1, 0, 2))


# speedup vs baseline: 811.2086x; 1.0629x over previous
import jax
import jax.numpy as jnp
from jax.experimental import pallas as pl
from jax.experimental.pallas import tpu as pltpu

_TC = 128


def _bilstm_kernel(seq_ref, xf_ref, xb_ref,
                   wih_f_ref, whh_f_ref, b_f_ref,
                   wih_b_ref, whh_b_ref, b_b_ref,
                   outf_ref, outb_ref,
                   gf_ref, gb_ref, hf_ref, cf_ref, hb_ref, cb_ref):
    k = pl.program_id(0)
    nchunk = pl.num_programs(0)
    tc, bsz, d = xf_ref.shape
    h_dim = whh_f_ref.shape[0]

    @pl.when(k == 0)
    def _init():
        hf_ref[...] = jnp.zeros_like(hf_ref)
        cf_ref[...] = jnp.zeros_like(cf_ref)
        hb_ref[...] = jnp.zeros_like(hb_ref)
        cb_ref[...] = jnp.zeros_like(cb_ref)

    xf = xf_ref[...].reshape(tc * bsz, d)
    gf_ref[...] = (jnp.dot(xf, wih_f_ref[...], preferred_element_type=jnp.float32)
                   + b_f_ref[...]).reshape(tc, bsz, 4 * h_dim)
    xb = xb_ref[...].reshape(tc * bsz, d)
    gb_ref[...] = (jnp.dot(xb, wih_b_ref[...], preferred_element_type=jnp.float32)
                   + b_b_ref[...]).reshape(tc, bsz, 4 * h_dim)

    seq_b = seq_ref[...]
    whh_f = whh_f_ref[...]
    whh_b = whh_b_ref[...]

    t0f = k * tc
    t0b = (nchunk - 1 - k) * tc

    def cell(gate_row, h, c, whh):
        gates = gate_row + jnp.dot(h, whh, preferred_element_type=jnp.float32)
        sig = 0.5 * jnp.tanh(0.5 * gates[:, 0:3 * h_dim]) + 0.5
        ig = sig[:, 0:h_dim]
        fg = sig[:, h_dim:2 * h_dim]
        og = sig[:, 2 * h_dim:3 * h_dim]
        gg = jnp.tanh(gates[:, 3 * h_dim:4 * h_dim])
        c = fg * c + ig * gg
        h = og * jnp.tanh(c)
        return h, c

    def body(i, carry):
        hf, cf, hb, cb = carry
        hf, cf = cell(gf_ref[i], hf, cf, whh_f)
        outf_ref[i] = hf
        j = tc - 1 - i
        tb = t0b + j
        keep = tb < seq_b - 1
        hb_in = jnp.where(keep, hb, 0.0)
        cb_in = jnp.where(keep, cb, 0.0)
        hb, cb = cell(gb_ref[j], hb_in, cb_in, whh_b)
        outb_ref[j] = hb
        return hf, cf, hb, cb

    carry0 = (hf_ref[...], cf_ref[...], hb_ref[...], cb_ref[...])
    hf, cf, hb, cb = jax.lax.fori_loop(0, tc, body, carry0, unroll=8)
    hf_ref[...] = hf
    cf_ref[...] = cf
    hb_ref[...] = hb
    cb_ref[...] = cb


def kernel(inputs, seq_len, W_ih_f, W_hh_f, b_ih_f, b_hh_f,
           W_ih_b, W_hh_b, b_ih_b, b_hh_b):
    bsz, t_len, d = inputs.shape
    h_dim = W_hh_f.shape[1]
    tc = min(_TC, t_len)
    nchunk = t_len // tc
    assert t_len % tc == 0

    xT = jnp.transpose(inputs, (1, 0, 2))
    seq_b = jnp.broadcast_to(seq_len.astype(jnp.int32)[:, None], (bsz, h_dim))

    def perm(w):
        return jnp.concatenate([w[0:h_dim], w[h_dim:2 * h_dim],
                                w[3 * h_dim:4 * h_dim], w[2 * h_dim:3 * h_dim]],
                               axis=0)

    args = (seq_b, xT, xT,
            perm(W_ih_f).T, perm(W_hh_f).T, perm(b_ih_f + b_hh_f)[None, :],
            perm(W_ih_b).T, perm(W_hh_b).T, perm(b_ih_b + b_hh_b)[None, :])

    full2 = lambda k: (0, 0)
    in_specs = [
        pl.BlockSpec((bsz, h_dim), full2),
        pl.BlockSpec((tc, bsz, d), lambda k: (k, 0, 0)),
        pl.BlockSpec((tc, bsz, d), lambda k: (nchunk - 1 - k, 0, 0)),
        pl.BlockSpec((d, 4 * h_dim), full2),
        pl.BlockSpec((h_dim, 4 * h_dim), full2),
        pl.BlockSpec((1, 4 * h_dim), full2),
        pl.BlockSpec((d, 4 * h_dim), full2),
        pl.BlockSpec((h_dim, 4 * h_dim), full2),
        pl.BlockSpec((1, 4 * h_dim), full2),
    ]
    out_specs = [
        pl.BlockSpec((tc, bsz, h_dim), lambda k: (k, 0, 0)),
        pl.BlockSpec((tc, bsz, h_dim), lambda k: (nchunk - 1 - k, 0, 0)),
    ]
    outf, outb = pl.pallas_call(
        _bilstm_kernel,
        grid=(nchunk,),
        in_specs=in_specs,
        out_specs=out_specs,
        out_shape=[jax.ShapeDtypeStruct((t_len, bsz, h_dim), jnp.float32)] * 2,
        scratch_shapes=[
            pltpu.VMEM((tc, bsz, 4 * h_dim), jnp.float32),
            pltpu.VMEM((tc, bsz, 4 * h_dim), jnp.float32),
            pltpu.VMEM((bsz, h_dim), jnp.float32),
            pltpu.VMEM((bsz, h_dim), jnp.float32),
            pltpu.VMEM((bsz, h_dim), jnp.float32),
            pltpu.VMEM((bsz, h_dim), jnp.float32),
        ],
    )(*args)
    out = jnp.concatenate([outf, outb], axis=-1)
    out = jnp.transpose(out, (1, 0, 2))
    mask = jnp.arange(t_len, dtype=jnp.int32)[None, :] < seq_len[:, None]
    return out * mask[..., None].astype(out.dtype)


# speedup vs baseline: 845.3796x; 1.0421x over previous
import jax
import jax.numpy as jnp
from jax.experimental import pallas as pl
from jax.experimental.pallas import tpu as pltpu

_TC = 128


def _bilstm_kernel(seq_ref, xf_ref, xb_ref,
                   wih_f_ref, whh_f_ref, b_f_ref,
                   wih_b_ref, whh_b_ref, b_b_ref,
                   out_ref,
                   gf_ref, gb_ref, hf_ref, cf_ref, hb_ref, cb_ref,
                   of_ref, ob_ref):
    k = pl.program_id(0)
    d = pl.program_id(1)
    nchunk = pl.num_programs(0)
    tc, bsz, dm = xf_ref.shape
    h_dim = whh_f_ref.shape[0]

    def masked_transpose(stage, t0):
        vals = jnp.swapaxes(stage, 0, 1)
        t_idx = t0 + jax.lax.broadcasted_iota(jnp.int32, (bsz, tc, h_dim), 1)
        return jnp.where(t_idx < seq_ref[...][:, None, :], vals, 0.0)

    @pl.when(d == 0)
    def _compute():
        @pl.when(k == 0)
        def _init():
            hf_ref[...] = jnp.zeros_like(hf_ref)
            cf_ref[...] = jnp.zeros_like(cf_ref)
            hb_ref[...] = jnp.zeros_like(hb_ref)
            cb_ref[...] = jnp.zeros_like(cb_ref)

        xf = xf_ref[...].reshape(tc * bsz, dm)
        gf_ref[...] = (jnp.dot(xf, wih_f_ref[...],
                               preferred_element_type=jnp.float32)
                       + b_f_ref[...]).reshape(tc, bsz, 4 * h_dim)
        xb = xb_ref[...].reshape(tc * bsz, dm)
        gb_ref[...] = (jnp.dot(xb, wih_b_ref[...],
                               preferred_element_type=jnp.float32)
                       + b_b_ref[...]).reshape(tc, bsz, 4 * h_dim)

        seq_b = seq_ref[...]
        whh_f = whh_f_ref[...]
        whh_b = whh_b_ref[...]
        t0b = (nchunk - 1 - k) * tc

        def cell(gate_row, h, c, whh):
            gates = gate_row + jnp.dot(h, whh,
                                       preferred_element_type=jnp.float32)
            sig = 0.5 * jnp.tanh(0.5 * gates[:, 0:3 * h_dim]) + 0.5
            ig = sig[:, 0:h_dim]
            fg = sig[:, h_dim:2 * h_dim]
            og = sig[:, 2 * h_dim:3 * h_dim]
            gg = jnp.tanh(gates[:, 3 * h_dim:4 * h_dim])
            c = fg * c + ig * gg
            h = og * jnp.tanh(c)
            return h, c

        def body(i, carry):
            hf, cf, hb, cb = carry
            hf, cf = cell(gf_ref[i], hf, cf, whh_f)
            of_ref[i] = hf
            j = tc - 1 - i
            tb = t0b + j
            keep = tb < seq_b - 1
            hb_in = jnp.where(keep, hb, 0.0)
            cb_in = jnp.where(keep, cb, 0.0)
            hb, cb = cell(gb_ref[j], hb_in, cb_in, whh_b)
            ob_ref[j] = hb
            return hf, cf, hb, cb

        carry0 = (hf_ref[...], cf_ref[...], hb_ref[...], cb_ref[...])
        hf, cf, hb, cb = jax.lax.fori_loop(0, tc, body, carry0, unroll=8)
        hf_ref[...] = hf
        cf_ref[...] = cf
        hb_ref[...] = hb
        cb_ref[...] = cb

        out_ref[...] = masked_transpose(of_ref[...], k * tc)

    @pl.when(d == 1)
    def _flush_backward():
        out_ref[...] = masked_transpose(ob_ref[...], (nchunk - 1 - k) * tc)


def kernel(inputs, seq_len, W_ih_f, W_hh_f, b_ih_f, b_hh_f,
           W_ih_b, W_hh_b, b_ih_b, b_hh_b):
    bsz, t_len, dm = inputs.shape
    h_dim = W_hh_f.shape[1]
    tc = min(_TC, t_len)
    nchunk = t_len // tc
    assert t_len % tc == 0

    xT = jnp.transpose(inputs, (1, 0, 2))
    seq_b = jnp.broadcast_to(seq_len.astype(jnp.int32)[:, None], (bsz, h_dim))

    def perm(w):
        return jnp.concatenate([w[0:h_dim], w[h_dim:2 * h_dim],
                                w[3 * h_dim:4 * h_dim], w[2 * h_dim:3 * h_dim]],
                               axis=0)

    args = (seq_b, xT, xT,
            perm(W_ih_f).T, perm(W_hh_f).T, perm(b_ih_f + b_hh_f)[None, :],
            perm(W_ih_b).T, perm(W_hh_b).T, perm(b_ih_b + b_hh_b)[None, :])

    full2 = lambda k, d: (0, 0)
    in_specs = [
        pl.BlockSpec((bsz, h_dim), full2),
        pl.BlockSpec((tc, bsz, dm), lambda k, d: (k, 0, 0)),
        pl.BlockSpec((tc, bsz, dm), lambda k, d: (nchunk - 1 - k, 0, 0)),
        pl.BlockSpec((dm, 4 * h_dim), full2),
        pl.BlockSpec((h_dim, 4 * h_dim), full2),
        pl.BlockSpec((1, 4 * h_dim), full2),
        pl.BlockSpec((dm, 4 * h_dim), full2),
        pl.BlockSpec((h_dim, 4 * h_dim), full2),
        pl.BlockSpec((1, 4 * h_dim), full2),
    ]
    out_spec = pl.BlockSpec(
        (bsz, tc, h_dim),
        lambda k, d: (0, jnp.where(d == 0, k, nchunk - 1 - k), d))
    out = pl.pallas_call(
        _bilstm_kernel,
        grid=(nchunk, 2),
        in_specs=in_specs,
        out_specs=out_spec,
        out_shape=jax.ShapeDtypeStruct((bsz, t_len, 2 * h_dim), jnp.float32),
        scratch_shapes=[
            pltpu.VMEM((tc, bsz, 4 * h_dim), jnp.float32),
            pltpu.VMEM((tc, bsz, 4 * h_dim), jnp.float32),
            pltpu.VMEM((bsz, h_dim), jnp.float32),
            pltpu.VMEM((bsz, h_dim), jnp.float32),
            pltpu.VMEM((bsz, h_dim), jnp.float32),
            pltpu.VMEM((bsz, h_dim), jnp.float32),
            pltpu.VMEM((tc, bsz, h_dim), jnp.float32),
            pltpu.VMEM((tc, bsz, h_dim), jnp.float32),
        ],
    )(*args)
    return out


# speedup vs baseline: 896.2224x; 1.0601x over previous
import jax
import jax.numpy as jnp
from jax.experimental import pallas as pl
from jax.experimental.pallas import tpu as pltpu

_TC = 128


def _bilstm_kernel(seq_ref, xf_ref, xb_ref,
                   wih_f_ref, whh_f_ref, b_f_ref,
                   wih_b_ref, whh_b_ref, b_b_ref,
                   out_ref,
                   gf_ref, gb_ref, hf_ref, cf_ref, hb_ref, cb_ref,
                   of_ref, ob_ref):
    k = pl.program_id(0)
    d = pl.program_id(1)
    nchunk = pl.num_programs(0)
    bsz, tc, dm = xf_ref.shape
    h_dim = whh_f_ref.shape[0]

    def masked_transpose(stage, t0):
        vals = jnp.swapaxes(stage, 0, 1)
        t_idx = t0 + jax.lax.broadcasted_iota(jnp.int32, (bsz, tc, h_dim), 1)
        return jnp.where(t_idx < seq_ref[...][:, None, :], vals, 0.0)

    @pl.when(d == 0)
    def _compute():
        @pl.when(k == 0)
        def _init():
            hf_ref[...] = jnp.zeros_like(hf_ref)
            cf_ref[...] = jnp.zeros_like(cf_ref)
            hb_ref[...] = jnp.zeros_like(hb_ref)
            cb_ref[...] = jnp.zeros_like(cb_ref)

        xf = jnp.swapaxes(xf_ref[...], 0, 1).reshape(tc * bsz, dm)
        gf_ref[...] = (jnp.dot(xf, wih_f_ref[...],
                               preferred_element_type=jnp.float32)
                       + b_f_ref[...]).reshape(tc, bsz, 4 * h_dim)
        xb = jnp.swapaxes(xb_ref[...], 0, 1).reshape(tc * bsz, dm)
        gb_ref[...] = (jnp.dot(xb, wih_b_ref[...],
                               preferred_element_type=jnp.float32)
                       + b_b_ref[...]).reshape(tc, bsz, 4 * h_dim)

        seq_b = seq_ref[...]
        whh_f = whh_f_ref[...]
        whh_b = whh_b_ref[...]
        t0b = (nchunk - 1 - k) * tc

        def cell(gate_row, h, c, whh):
            gates = gate_row + jnp.dot(h, whh,
                                       preferred_element_type=jnp.float32)
            sig = 0.5 * jnp.tanh(0.5 * gates[:, 0:3 * h_dim]) + 0.5
            ig = sig[:, 0:h_dim]
            fg = sig[:, h_dim:2 * h_dim]
            og = sig[:, 2 * h_dim:3 * h_dim]
            gg = jnp.tanh(gates[:, 3 * h_dim:4 * h_dim])
            c = fg * c + ig * gg
            h = og * jnp.tanh(c)
            return h, c

        def body(i, carry):
            hf, cf, hb, cb = carry
            hf, cf = cell(gf_ref[i], hf, cf, whh_f)
            of_ref[i] = hf
            j = tc - 1 - i
            tb = t0b + j
            keep = tb < seq_b - 1
            hb_in = jnp.where(keep, hb, 0.0)
            cb_in = jnp.where(keep, cb, 0.0)
            hb, cb = cell(gb_ref[j], hb_in, cb_in, whh_b)
            ob_ref[j] = hb
            return hf, cf, hb, cb

        carry0 = (hf_ref[...], cf_ref[...], hb_ref[...], cb_ref[...])
        hf, cf, hb, cb = jax.lax.fori_loop(0, tc, body, carry0, unroll=8)
        hf_ref[...] = hf
        cf_ref[...] = cf
        hb_ref[...] = hb
        cb_ref[...] = cb

        out_ref[...] = masked_transpose(of_ref[...], k * tc)

    @pl.when(d == 1)
    def _flush_backward():
        out_ref[...] = masked_transpose(ob_ref[...], (nchunk - 1 - k) * tc)


def kernel(inputs, seq_len, W_ih_f, W_hh_f, b_ih_f, b_hh_f,
           W_ih_b, W_hh_b, b_ih_b, b_hh_b):
    bsz, t_len, dm = inputs.shape
    h_dim = W_hh_f.shape[1]
    tc = min(_TC, t_len)
    nchunk = t_len // tc
    assert t_len % tc == 0

    seq_b = jnp.broadcast_to(seq_len.astype(jnp.int32)[:, None], (bsz, h_dim))

    def perm(w):
        return jnp.concatenate([w[0:h_dim], w[h_dim:2 * h_dim],
                                w[3 * h_dim:4 * h_dim], w[2 * h_dim:3 * h_dim]],
                               axis=0)

    args = (seq_b, inputs, inputs,
            perm(W_ih_f).T, perm(W_hh_f).T, perm(b_ih_f + b_hh_f)[None, :],
            perm(W_ih_b).T, perm(W_hh_b).T, perm(b_ih_b + b_hh_b)[None, :])

    full2 = lambda k, d: (0, 0)
    in_specs = [
        pl.BlockSpec((bsz, h_dim), full2),
        pl.BlockSpec((bsz, tc, dm), lambda k, d: (0, k, 0)),
        pl.BlockSpec((bsz, tc, dm), lambda k, d: (0, nchunk - 1 - k, 0)),
        pl.BlockSpec((dm, 4 * h_dim), full2),
        pl.BlockSpec((h_dim, 4 * h_dim), full2),
        pl.BlockSpec((1, 4 * h_dim), full2),
        pl.BlockSpec((dm, 4 * h_dim), full2),
        pl.BlockSpec((h_dim, 4 * h_dim), full2),
        pl.BlockSpec((1, 4 * h_dim), full2),
    ]
    out_spec = pl.BlockSpec(
        (bsz, tc, h_dim),
        lambda k, d: (0, jnp.where(d == 0, k, nchunk - 1 - k), d))
    out = pl.pallas_call(
        _bilstm_kernel,
        grid=(nchunk, 2),
        in_specs=in_specs,
        out_specs=out_spec,
        out_shape=jax.ShapeDtypeStruct((bsz, t_len, 2 * h_dim), jnp.float32),
        scratch_shapes=[
            pltpu.VMEM((tc, bsz, 4 * h_dim), jnp.float32),
            pltpu.VMEM((tc, bsz, 4 * h_dim), jnp.float32),
            pltpu.VMEM((bsz, h_dim), jnp.float32),
            pltpu.VMEM((bsz, h_dim), jnp.float32),
            pltpu.VMEM((bsz, h_dim), jnp.float32),
            pltpu.VMEM((bsz, h_dim), jnp.float32),
            pltpu.VMEM((tc, bsz, h_dim), jnp.float32),
            pltpu.VMEM((tc, bsz, h_dim), jnp.float32),
        ],
    )(*args)
    return out


# speedup vs baseline: 897.4559x; 1.0014x over previous
import jax
import jax.numpy as jnp
from jax.experimental import pallas as pl
from jax.experimental.pallas import tpu as pltpu

_TC = 128


def _bilstm_kernel(seq_ref, xf_ref, xb_ref,
                   wih_f_ref, whh_f_ref, b_f_ref,
                   wih_b_ref, whh_b_ref, b_b_ref,
                   out_ref,
                   gf_ref, gb_ref, hf_ref, cf_ref, hb_ref, cb_ref,
                   of_ref, ob_ref):
    k = pl.program_id(0)
    d = pl.program_id(1)
    nchunk = pl.num_programs(0)
    bsz, tc, dm = xf_ref.shape
    h_dim = whh_f_ref.shape[0]

    def masked_transpose(stage, t0):
        vals = jnp.swapaxes(stage, 0, 1)
        t_idx = t0 + jax.lax.broadcasted_iota(jnp.int32, (bsz, tc, h_dim), 1)
        return jnp.where(t_idx < seq_ref[...][:, None, :], vals, 0.0)

    @pl.when(d == 0)
    def _compute():
        @pl.when(k == 0)
        def _init():
            hf_ref[...] = jnp.zeros_like(hf_ref)
            cf_ref[...] = jnp.zeros_like(cf_ref)
            hb_ref[...] = jnp.zeros_like(hb_ref)
            cb_ref[...] = jnp.zeros_like(cb_ref)

        xf = jnp.swapaxes(xf_ref[...], 0, 1).reshape(tc * bsz, dm)
        gf_ref[...] = (jnp.dot(xf.astype(jnp.bfloat16), wih_f_ref[...],
                               preferred_element_type=jnp.float32)
                       + b_f_ref[...]).reshape(tc, bsz, 4 * h_dim)
        xb = jnp.swapaxes(xb_ref[...], 0, 1).reshape(tc * bsz, dm)
        gb_ref[...] = (jnp.dot(xb.astype(jnp.bfloat16), wih_b_ref[...],
                               preferred_element_type=jnp.float32)
                       + b_b_ref[...]).reshape(tc, bsz, 4 * h_dim)

        seq_b = seq_ref[...]
        whh_f = whh_f_ref[...]
        whh_b = whh_b_ref[...]
        t0b = (nchunk - 1 - k) * tc

        def cell(gate_row, h, c, whh):
            gates = gate_row + jnp.dot(h.astype(jnp.bfloat16), whh,
                                       preferred_element_type=jnp.float32)
            sig = 0.5 * jnp.tanh(0.5 * gates[:, 0:3 * h_dim]) + 0.5
            ig = sig[:, 0:h_dim]
            fg = sig[:, h_dim:2 * h_dim]
            og = sig[:, 2 * h_dim:3 * h_dim]
            gg = jnp.tanh(gates[:, 3 * h_dim:4 * h_dim])
            c = fg * c + ig * gg
            h = og * jnp.tanh(c)
            return h, c

        def body(i, carry):
            hf, cf, hb, cb = carry
            hf, cf = cell(gf_ref[i], hf, cf, whh_f)
            of_ref[i] = hf
            j = tc - 1 - i
            tb = t0b + j
            keep = tb < seq_b - 1
            hb_in = jnp.where(keep, hb, 0.0)
            cb_in = jnp.where(keep, cb, 0.0)
            hb, cb = cell(gb_ref[j], hb_in, cb_in, whh_b)
            ob_ref[j] = hb
            return hf, cf, hb, cb

        carry0 = (hf_ref[...], cf_ref[...], hb_ref[...], cb_ref[...])
        hf, cf, hb, cb = jax.lax.fori_loop(0, tc, body, carry0, unroll=8)
        hf_ref[...] = hf
        cf_ref[...] = cf
        hb_ref[...] = hb
        cb_ref[...] = cb

        out_ref[...] = masked_transpose(of_ref[...], k * tc)

    @pl.when(d == 1)
    def _flush_backward():
        out_ref[...] = masked_transpose(ob_ref[...], (nchunk - 1 - k) * tc)


def kernel(inputs, seq_len, W_ih_f, W_hh_f, b_ih_f, b_hh_f,
           W_ih_b, W_hh_b, b_ih_b, b_hh_b):
    bsz, t_len, dm = inputs.shape
    h_dim = W_hh_f.shape[1]
    tc = min(_TC, t_len)
    nchunk = t_len // tc
    assert t_len % tc == 0

    seq_b = jnp.broadcast_to(seq_len.astype(jnp.int32)[:, None], (bsz, h_dim))

    def perm(w):
        return jnp.concatenate([w[0:h_dim], w[h_dim:2 * h_dim],
                                w[3 * h_dim:4 * h_dim], w[2 * h_dim:3 * h_dim]],
                               axis=0)

    bf = lambda w: w.astype(jnp.bfloat16)
    args = (seq_b, inputs, inputs,
            bf(perm(W_ih_f).T), bf(perm(W_hh_f).T),
            perm(b_ih_f + b_hh_f)[None, :],
            bf(perm(W_ih_b).T), bf(perm(W_hh_b).T),
            perm(b_ih_b + b_hh_b)[None, :])

    full2 = lambda k, d: (0, 0)
    in_specs = [
        pl.BlockSpec((bsz, h_dim), full2),
        pl.BlockSpec((bsz, tc, dm), lambda k, d: (0, k, 0)),
        pl.BlockSpec((bsz, tc, dm), lambda k, d: (0, nchunk - 1 - k, 0)),
        pl.BlockSpec((dm, 4 * h_dim), full2),
        pl.BlockSpec((h_dim, 4 * h_dim), full2),
        pl.BlockSpec((1, 4 * h_dim), full2),
        pl.BlockSpec((dm, 4 * h_dim), full2),
        pl.BlockSpec((h_dim, 4 * h_dim), full2),
        pl.BlockSpec((1, 4 * h_dim), full2),
    ]
    out_spec = pl.BlockSpec(
        (bsz, tc, h_dim),
        lambda k, d: (0, jnp.where(d == 0, k, nchunk - 1 - k), d))
    out = pl.pallas_call(
        _bilstm_kernel,
        grid=(nchunk, 2),
        in_specs=in_specs,
        out_specs=out_spec,
        out_shape=jax.ShapeDtypeStruct((bsz, t_len, 2 * h_dim), jnp.float32),
        scratch_shapes=[
            pltpu.VMEM((tc, bsz, 4 * h_dim), jnp.float32),
            pltpu.VMEM((tc, bsz, 4 * h_dim), jnp.float32),
            pltpu.VMEM((bsz, h_dim), jnp.float32),
            pltpu.VMEM((bsz, h_dim), jnp.float32),
            pltpu.VMEM((bsz, h_dim), jnp.float32),
            pltpu.VMEM((bsz, h_dim), jnp.float32),
            pltpu.VMEM((tc, bsz, h_dim), jnp.float32),
            pltpu.VMEM((tc, bsz, h_dim), jnp.float32),
        ],
    )(*args)
    return out


# speedup vs baseline: 923.6908x; 1.0292x over previous
import jax
import jax.numpy as jnp
from jax.experimental import pallas as pl
from jax.experimental.pallas import tpu as pltpu

_TC = 128


def _bilstm_kernel(seq_ref, xf_ref, xb_ref,
                   wih_f_ref, whh_f_ref, b_f_ref,
                   wih_b_ref, whh_b_ref, b_b_ref,
                   out_ref,
                   gf_ref, gb_ref, hf_ref, cf_ref, hb_ref, cb_ref,
                   of_ref, ob_ref):
    k = pl.program_id(0)
    d = pl.program_id(1)
    nchunk = pl.num_programs(0)
    bsz, tc, dm = xf_ref.shape
    h_dim = whh_f_ref.shape[0]

    def masked_transpose(stage, t0):
        vals = jnp.swapaxes(stage, 0, 1)
        t_idx = t0 + jax.lax.broadcasted_iota(jnp.int32, (bsz, tc, h_dim), 1)
        return jnp.where(t_idx < seq_ref[...][:, None, :], vals, 0.0)

    @pl.when(d == 0)
    def _compute():
        @pl.when(k == 0)
        def _init():
            hf_ref[...] = jnp.zeros_like(hf_ref)
            cf_ref[...] = jnp.zeros_like(cf_ref)
            hb_ref[...] = jnp.zeros_like(hb_ref)
            cb_ref[...] = jnp.zeros_like(cb_ref)

        xf = jnp.swapaxes(xf_ref[...], 0, 1).reshape(tc * bsz, dm)
        gf_ref[...] = (jnp.dot(xf.astype(jnp.bfloat16), wih_f_ref[...],
                               preferred_element_type=jnp.float32)
                       + b_f_ref[...]).reshape(tc, bsz, 4 * h_dim)
        xb = jnp.swapaxes(xb_ref[...], 0, 1).reshape(tc * bsz, dm)
        gb_ref[...] = (jnp.dot(xb.astype(jnp.bfloat16), wih_b_ref[...],
                               preferred_element_type=jnp.float32)
                       + b_b_ref[...]).reshape(tc, bsz, 4 * h_dim)

        seq_b = seq_ref[...]
        whh_f = whh_f_ref[...]
        whh_b = whh_b_ref[...]
        t0b = (nchunk - 1 - k) * tc

        def cell(gate_row, h, c, whh):
            gates = gate_row + jnp.dot(h.astype(jnp.bfloat16), whh,
                                       preferred_element_type=jnp.float32)
            sig = 0.5 * jnp.tanh(0.5 * gates[:, 0:3 * h_dim]) + 0.5
            ig = sig[:, 0:h_dim]
            fg = sig[:, h_dim:2 * h_dim]
            og = sig[:, 2 * h_dim:3 * h_dim]
            gg = jnp.tanh(gates[:, 3 * h_dim:4 * h_dim])
            c = fg * c + ig * gg
            h = og * jnp.tanh(c)
            return h, c

        def body(i, carry):
            hf, cf, hb, cb = carry
            hf, cf = cell(gf_ref[i], hf, cf, whh_f)
            of_ref[i] = hf
            j = tc - 1 - i
            tb = t0b + j
            keep = tb < seq_b - 1
            hb_in = jnp.where(keep, hb, 0.0)
            cb_in = jnp.where(keep, cb, 0.0)
            hb, cb = cell(gb_ref[j], hb_in, cb_in, whh_b)
            ob_ref[j] = hb
            return hf, cf, hb, cb

        carry0 = (hf_ref[...], cf_ref[...], hb_ref[...], cb_ref[...])
        hf, cf, hb, cb = jax.lax.fori_loop(0, tc, body, carry0, unroll=16)
        hf_ref[...] = hf
        cf_ref[...] = cf
        hb_ref[...] = hb
        cb_ref[...] = cb

        out_ref[...] = masked_transpose(of_ref[...], k * tc)

    @pl.when(d == 1)
    def _flush_backward():
        out_ref[...] = masked_transpose(ob_ref[...], (nchunk - 1 - k) * tc)


def kernel(inputs, seq_len, W_ih_f, W_hh_f, b_ih_f, b_hh_f,
           W_ih_b, W_hh_b, b_ih_b, b_hh_b):
    bsz, t_len, dm = inputs.shape
    h_dim = W_hh_f.shape[1]
    tc = min(_TC, t_len)
    nchunk = t_len // tc
    assert t_len % tc == 0

    seq_b = jnp.broadcast_to(seq_len.astype(jnp.int32)[:, None], (bsz, h_dim))

    def perm(w):
        return jnp.concatenate([w[0:h_dim], w[h_dim:2 * h_dim],
                                w[3 * h_dim:4 * h_dim], w[2 * h_dim:3 * h_dim]],
                               axis=0)

    bf = lambda w: w.astype(jnp.bfloat16)
    args = (seq_b, inputs, inputs,
            bf(perm(W_ih_f).T), bf(perm(W_hh_f).T),
            perm(b_ih_f + b_hh_f)[None, :],
            bf(perm(W_ih_b).T), bf(perm(W_hh_b).T),
            perm(b_ih_b + b_hh_b)[None, :])

    full2 = lambda k, d: (0, 0)
    in_specs = [
        pl.BlockSpec((bsz, h_dim), full2),
        pl.BlockSpec((bsz, tc, dm), lambda k, d: (0, k, 0)),
        pl.BlockSpec((bsz, tc, dm), lambda k, d: (0, nchunk - 1 - k, 0)),
        pl.BlockSpec((dm, 4 * h_dim), full2),
        pl.BlockSpec((h_dim, 4 * h_dim), full2),
        pl.BlockSpec((1, 4 * h_dim), full2),
        pl.BlockSpec((dm, 4 * h_dim), full2),
        pl.BlockSpec((h_dim, 4 * h_dim), full2),
        pl.BlockSpec((1, 4 * h_dim), full2),
    ]
    out_spec = pl.BlockSpec(
        (bsz, tc, h_dim),
        lambda k, d: (0, jnp.where(d == 0, k, nchunk - 1 - k), d))
    out = pl.pallas_call(
        _bilstm_kernel,
        grid=(nchunk, 2),
        in_specs=in_specs,
        out_specs=out_spec,
        out_shape=jax.ShapeDtypeStruct((bsz, t_len, 2 * h_dim), jnp.float32),
        scratch_shapes=[
            pltpu.VMEM((tc, bsz, 4 * h_dim), jnp.float32),
            pltpu.VMEM((tc, bsz, 4 * h_dim), jnp.float32),
            pltpu.VMEM((bsz, h_dim), jnp.float32),
            pltpu.VMEM((bsz, h_dim), jnp.float32),
            pltpu.VMEM((bsz, h_dim), jnp.float32),
            pltpu.VMEM((bsz, h_dim), jnp.float32),
            pltpu.VMEM((tc, bsz, h_dim), jnp.float32),
            pltpu.VMEM((tc, bsz, h_dim), jnp.float32),
        ],
    )(*args)
    return out
